# baseline (speedup 1.0000x reference)
import functools

import jax
import jax.numpy as jnp
from jax import lax
from jax.experimental import pallas as pl
from jax.experimental.pallas import tpu as pltpu
from jax.experimental.pallas import tpu_sc as plsc

G_GRP = 64
F_FEAT = 128
HW = 4096
NUM_CELLS = G_GRP * HW
N_INS = 200000
N_PAD = 200064
NW = 32
CPW = NUM_CELLS // NW
CHUNK = 2000
NCHUNK = N_INS // CHUNK
GROWS = 128
NGC_H = HW // GROWS
HALF = NUM_CELLS // 2


def _tc_cells(grp, gx, gy):
    def body(g_ref, x_ref, y_ref, o_ref):
        o_ref[...] = (g_ref[...] << 12) | (y_ref[...] << 6) | x_ref[...]

    return pl.pallas_call(
        body,
        out_shape=jax.ShapeDtypeStruct((N_PAD // 128, 128), jnp.int32),
    )(grp.reshape(N_PAD // 128, 128),
      gx.reshape(N_PAD // 128, 128),
      gy.reshape(N_PAD // 128, 128)).reshape(N_PAD)


def _gather_ring(w_ref, w_off, ins_hbm, g_out, g_base, idx_bufs, row_bufs,
                 sems, lanes):
    def build_idx(c, buf):
        def idx_body(j, _):
            w = w_ref[pl.ds(w_off + c * GROWS + j * 16, 16)]
            m = w > 0
            gc = g_base + c * GROWS + j * 16 + lanes
            junk = gc & 131071
            idx_bufs[buf][pl.ds(j * 16, 16)] = jnp.where(m, w - 1, junk)
            return 0
        lax.fori_loop(0, GROWS // 16, idx_body, 0, unroll=4)

    def start_gather(buf):
        pltpu.async_copy(ins_hbm.at[idx_bufs[buf]], row_bufs[buf], sems[buf])

    def wait_gather(buf):
        pltpu.make_async_copy(ins_hbm.at[idx_bufs[buf]], row_bufs[buf],
                              sems[buf]).wait()

    build_idx(0, 0)
    start_gather(0)

    def gather_step(c, cur):
        nxt = c + 1

        @pl.when(nxt < NGC_H)
        def _():
            build_idx(nxt, 1 - cur)
            start_gather(1 - cur)

        wait_gather(cur)
        pltpu.sync_copy(row_bufs[cur],
                        g_out.at[pl.ds(g_base + c * GROWS, GROWS)])

    def gather_pair(p, _):
        gather_step(2 * p, 0)
        gather_step(2 * p + 1, 1)
        return 0

    lax.fori_loop(0, NGC_H // 2, gather_pair, 0)


def _sc_build(cells, ins_feats):
    mesh = plsc.VectorSubcoreMesh(core_axis_name="c", subcore_axis_name="s")

    @functools.partial(
        pl.kernel,
        mesh=mesh,
        compiler_params=pltpu.CompilerParams(needs_layout_passes=False),
        out_type=[
            jax.ShapeDtypeStruct((NUM_CELLS,), jnp.int32),
            jax.ShapeDtypeStruct((HALF, F_FEAT), jnp.float32),
        ],
        scratch_types=[
            pltpu.VMEM((CPW,), jnp.int32),
            pltpu.VMEM((CHUNK,), jnp.int32),
            pltpu.VMEM((CHUNK,), jnp.int32),
            pltpu.VMEM((GROWS,), jnp.int32),
            pltpu.VMEM((GROWS,), jnp.int32),
            pltpu.VMEM((GROWS, F_FEAT), jnp.float32),
            pltpu.VMEM((GROWS, F_FEAT), jnp.float32),
            pltpu.SemaphoreType.DMA,
            pltpu.SemaphoreType.DMA,
            pltpu.SemaphoreType.DMA,
        ],
    )
    def sc_kernel(cells_hbm, ins_hbm, w_out, g_out,
                  w_v, cell_v0, cell_v1, idx_v0, idx_v1,
                  rows_v0, rows_v1, sem0, sem1, semw):
        wid = lax.axis_index("s") * 2 + lax.axis_index("c")
        lanes = lax.iota(jnp.int32, 16)
        zeros16 = jnp.zeros((16,), jnp.int32)
        sems = (sem0, sem1)
        cell_bufs = (cell_v0, cell_v1)

        def initb(i, _):
            w_v[pl.ds(i * 16, 16)] = zeros16
            return 0
        lax.fori_loop(0, CPW // 16, initb, 0, unroll=4)

        pltpu.async_copy(cells_hbm.at[pl.ds(0, CHUNK)], cell_v0, sem0)

        def chunk_step(ci, cur):
            ins_base = ci * CHUNK
            nxt = ci + 1

            @pl.when(nxt < NCHUNK)
            def _():
                pltpu.async_copy(
                    cells_hbm.at[pl.ds(nxt * CHUNK, CHUNK)],
                    cell_bufs[1 - cur], sems[1 - cur])

            pltpu.make_async_copy(
                cells_hbm.at[pl.ds(0, CHUNK)], cell_bufs[cur],
                sems[cur]).wait()

            def scan_body(j, _):
                cell = cell_bufs[cur][pl.ds(j * 16, 16)]
                gi = cell >> 12
                inr = (gi & 31) == wid
                local = (cell & (HW - 1)) | ((cell >> 17) << 12)
                val = (ins_base + j * 16 + 1) + lanes
                plsc.store_scatter(w_v, [local], val, mask=inr)
                return 0

            lax.fori_loop(0, CHUNK // 16, scan_body, 0, unroll=4)

        def chunk_pair(p, _):
            chunk_step(2 * p, 0)
            chunk_step(2 * p + 1, 1)
            return 0

        lax.fori_loop(0, NCHUNK // 2, chunk_pair, 0)

        pltpu.async_copy(w_v.at[pl.ds(0, HW)],
                         w_out.at[pl.ds(wid * HW, HW)], semw)
        pltpu.async_copy(w_v.at[pl.ds(HW, HW)],
                         w_out.at[pl.ds((wid + 32) * HW, HW)], semw)

        _gather_ring(w_v, 0, ins_hbm, g_out, wid * HW,
                     (idx_v0, idx_v1), (rows_v0, rows_v1), sems, lanes)

        pltpu.make_async_copy(w_v.at[pl.ds(0, HW)],
                              w_out.at[pl.ds(wid * HW, HW)], semw).wait()
        pltpu.make_async_copy(w_v.at[pl.ds(0, HW)],
                              w_out.at[pl.ds(wid * HW, HW)], semw).wait()

    return sc_kernel(cells, ins_feats)


def _sc_gather_hi(w, ins_feats):
    mesh = plsc.VectorSubcoreMesh(core_axis_name="c", subcore_axis_name="s")

    @functools.partial(
        pl.kernel,
        mesh=mesh,
        compiler_params=pltpu.CompilerParams(needs_layout_passes=False),
        out_type=jax.ShapeDtypeStruct((HALF, F_FEAT), jnp.float32),
        scratch_types=[
            pltpu.VMEM((HW,), jnp.int32),
            pltpu.VMEM((GROWS,), jnp.int32),
            pltpu.VMEM((GROWS,), jnp.int32),
            pltpu.VMEM((GROWS, F_FEAT), jnp.float32),
            pltpu.VMEM((GROWS, F_FEAT), jnp.float32),
            pltpu.SemaphoreType.DMA,
            pltpu.SemaphoreType.DMA,
        ],
    )
    def sc_kernel(w_hbm, ins_hbm, g_out,
                  w_v, idx_v0, idx_v1, rows_v0, rows_v1, sem0, sem1):
        wid = lax.axis_index("s") * 2 + lax.axis_index("c")
        lanes = lax.iota(jnp.int32, 16)
        pltpu.sync_copy(w_hbm.at[pl.ds((wid + 32) * HW, HW)], w_v)
        _gather_ring(w_v, 0, ins_hbm, g_out, wid * HW,
                     (idx_v0, idx_v1), (rows_v0, rows_v1), (sem0, sem1),
                     lanes)

    return sc_kernel(w, ins_feats)


GPB = 4
NBL = 32 // GPB


def _tc_select_kernel(w_ref, g_ref, f_ref, o_ref):
    for k in range(GPB):
        mask = w_ref[k] > 0
        t = lax.transpose(g_ref[k], (1, 0))
        o_ref[k] = jnp.where(mask, t, f_ref[k])


def _tc_select_lo(w, g_lo, f):
    return pl.pallas_call(
        _tc_select_kernel,
        grid=(NBL,),
        in_specs=[
            pl.BlockSpec((GPB, 1, HW), lambda i: (i, 0, 0)),
            pl.BlockSpec((GPB, HW, F_FEAT), lambda i: (i, 0, 0)),
            pl.BlockSpec((GPB, F_FEAT, HW), lambda i: (i, 0, 0)),
        ],
        out_specs=pl.BlockSpec((GPB, F_FEAT, HW), lambda i: (i, 0, 0)),
        out_shape=jax.ShapeDtypeStruct((G_GRP, F_FEAT, HW), jnp.float32),
    )(w, g_lo, f)


def _tc_select_hi_kernel(p_ref, w_ref, g_ref, f_ref, o_ref):
    del p_ref
    _tc_select_kernel(w_ref, g_ref, f_ref, o_ref)


def _tc_select_hi(partial, w, g_hi, f):
    return pl.pallas_call(
        _tc_select_hi_kernel,
        grid=(NBL,),
        in_specs=[
            pl.BlockSpec(memory_space=pl.ANY),
            pl.BlockSpec((GPB, 1, HW), lambda i: (i + NBL, 0, 0)),
            pl.BlockSpec((GPB, HW, F_FEAT), lambda i: (i, 0, 0)),
            pl.BlockSpec((GPB, F_FEAT, HW), lambda i: (i + NBL, 0, 0)),
        ],
        out_specs=pl.BlockSpec((GPB, F_FEAT, HW), lambda i: (i + NBL, 0, 0)),
        out_shape=jax.ShapeDtypeStruct((G_GRP, F_FEAT, HW), jnp.float32),
        input_output_aliases={0: 0},
    )(partial, w, g_hi, f)


def kernel(feat_map, grp_ids, grid_ids, ins_feats):
    pad = N_PAD - N_INS
    grp_p = jnp.pad(grp_ids, (0, pad))
    gx_p = jnp.pad(grid_ids[:, 0], (0, pad))
    gy_p = jnp.pad(grid_ids[:, 1], (0, pad))
    cells = _tc_cells(grp_p, gx_p, gy_p)
    w, g_lo = _sc_build(cells, ins_feats)
    g_hi = _sc_gather_hi(w, ins_feats)
    w3 = w.reshape(G_GRP, 1, HW)
    f3 = feat_map.reshape(G_GRP, F_FEAT, HW)
    partial = _tc_select_lo(w3, g_lo.reshape(32, HW, F_FEAT), f3)
    out = _tc_select_hi(partial, w3, g_hi.reshape(32, HW, F_FEAT), f3)
    return out.reshape(feat_map.shape)


# speedup vs baseline: 1.1681x; 1.1681x over previous
import functools

import jax
import jax.numpy as jnp
from jax import lax
from jax.experimental import pallas as pl
from jax.experimental.pallas import tpu as pltpu
from jax.experimental.pallas import tpu_sc as plsc

G_GRP = 64
F_FEAT = 128
HW = 4096
NUM_CELLS = G_GRP * HW
N_INS = 200000
N_PAD = 200064
NW = 32
CPW = NUM_CELLS // NW
CHUNK = 4000
NCHUNK = N_INS // CHUNK
CAND_CAP = 2048
GROWS = 128
NGC = CPW // GROWS


def _tc_cells(grp, gx, gy):
    def body(g_ref, x_ref, y_ref, o_ref):
        o_ref[...] = (g_ref[...] << 12) | (y_ref[...] << 6) | x_ref[...]

    return pl.pallas_call(
        body,
        out_shape=jax.ShapeDtypeStruct((N_PAD // 128, 128), jnp.int32),
    )(grp.reshape(N_PAD // 128, 128),
      gx.reshape(N_PAD // 128, 128),
      gy.reshape(N_PAD // 128, 128)).reshape(N_PAD)


def _sc_build(cells, ins_feats):
    mesh = plsc.VectorSubcoreMesh(core_axis_name="c", subcore_axis_name="s")

    @functools.partial(
        pl.kernel,
        mesh=mesh,
        compiler_params=pltpu.CompilerParams(needs_layout_passes=False),
        out_type=[
            jax.ShapeDtypeStruct((NUM_CELLS,), jnp.int32),
            jax.ShapeDtypeStruct((NUM_CELLS, F_FEAT), jnp.float32),
        ],
        scratch_types=[
            pltpu.VMEM((CPW,), jnp.int32),
            pltpu.VMEM((CHUNK,), jnp.int32),
            pltpu.VMEM((CHUNK,), jnp.int32),
            pltpu.VMEM((CAND_CAP,), jnp.int32),
            pltpu.VMEM((GROWS,), jnp.int32),
            pltpu.VMEM((GROWS,), jnp.int32),
            pltpu.VMEM((GROWS, F_FEAT), jnp.float32),
            pltpu.VMEM((GROWS, F_FEAT), jnp.float32),
            pltpu.SemaphoreType.DMA,
            pltpu.SemaphoreType.DMA,
            pltpu.SemaphoreType.DMA,
        ],
    )
    def sc_kernel(cells_hbm, ins_hbm, w_out, g_out,
                  w_v, cell_v0, cell_v1, cand_v, idx_v0, idx_v1,
                  rows_v0, rows_v1, sem0, sem1, semw):
        wid = lax.axis_index("s") * 2 + lax.axis_index("c")
        base_cell = wid * CPW
        lanes = lax.iota(jnp.int32, 16)
        zeros16 = jnp.zeros((16,), jnp.int32)
        sems = (sem0, sem1)
        cell_bufs = (cell_v0, cell_v1)
        idx_bufs = (idx_v0, idx_v1)
        row_bufs = (rows_v0, rows_v1)

        def initb(i, _):
            w_v[pl.ds(i * 16, 16)] = zeros16
            return 0
        lax.fori_loop(0, CPW // 16, initb, 0, unroll=4)

        pltpu.async_copy(cells_hbm.at[pl.ds(0, CHUNK)], cell_v0, sem0)

        def chunk_step(ci, cur):
            ins_base = ci * CHUNK
            nxt = ci + 1

            @pl.when(nxt < NCHUNK)
            def _():
                pltpu.async_copy(
                    cells_hbm.at[pl.ds(nxt * CHUNK, CHUNK)],
                    cell_bufs[1 - cur], sems[1 - cur])

            pltpu.make_async_copy(
                cells_hbm.at[pl.ds(0, CHUNK)], cell_bufs[cur],
                sems[cur]).wait()

            def scan_body(j, _):
                cell = cell_bufs[cur][pl.ds(j * 16, 16)]
                local = cell - base_cell
                inr = (local >= 0) & (local < CPW)
                val = (ins_base + j * 16 + 1) + lanes
                plsc.store_scatter(w_v, [local & (CPW - 1)], val, mask=inr)
                return 0

            lax.fori_loop(0, CHUNK // 16, scan_body, 0, unroll=8)

        def chunk_pair(p, _):
            chunk_step(2 * p, 0)
            chunk_step(2 * p + 1, 1)
            return 0

        lax.fori_loop(0, NCHUNK // 2, chunk_pair, 0)

        pltpu.async_copy(w_v, w_out.at[pl.ds(base_cell, CPW)], semw)

        def build_idx(c, buf):
            def idx_body(j, _):
                w = w_v[pl.ds(c * GROWS + j * 16, 16)]
                m = w > 0
                gc = base_cell + c * GROWS + j * 16 + lanes
                junk = gc & 131071
                idx_bufs[buf][pl.ds(j * 16, 16)] = jnp.where(m, w - 1, junk)
                return 0
            lax.fori_loop(0, GROWS // 16, idx_body, 0, unroll=4)

        def start_gather(c, buf):
            pltpu.async_copy(ins_hbm.at[idx_bufs[buf]], row_bufs[buf],
                             sems[buf])

        def wait_gather(buf):
            pltpu.make_async_copy(ins_hbm.at[idx_bufs[buf]], row_bufs[buf],
                                  sems[buf]).wait()

        build_idx(0, 0)
        start_gather(0, 0)

        def gather_step(c, cur):
            nxt = c + 1

            @pl.when(nxt < NGC)
            def _():
                build_idx(nxt, 1 - cur)
                start_gather(nxt, 1 - cur)

            wait_gather(cur)
            pltpu.sync_copy(row_bufs[cur],
                            g_out.at[pl.ds(base_cell + c * GROWS, GROWS)])

        def gather_pair(p, _):
            gather_step(2 * p, 0)
            gather_step(2 * p + 1, 1)
            return 0

        lax.fori_loop(0, NGC // 2, gather_pair, 0)
        pltpu.make_async_copy(w_v, w_out.at[pl.ds(base_cell, CPW)],
                              semw).wait()

    return sc_kernel(cells, ins_feats)


GPB = 4


def _tc_select_kernel(w_ref, g_ref, f_ref, o_ref):
    for k in range(GPB):
        mask = w_ref[k] > 0
        t = lax.transpose(g_ref[k], (1, 0))
        o_ref[k] = jnp.where(mask, t, f_ref[k])


def _tc_select(w, g, f):
    return pl.pallas_call(
        _tc_select_kernel,
        grid=(G_GRP // GPB,),
        in_specs=[
            pl.BlockSpec((GPB, 1, HW), lambda i: (i, 0, 0)),
            pl.BlockSpec((GPB, HW, F_FEAT), lambda i: (i, 0, 0)),
            pl.BlockSpec((GPB, F_FEAT, HW), lambda i: (i, 0, 0)),
        ],
        out_specs=pl.BlockSpec((GPB, F_FEAT, HW), lambda i: (i, 0, 0)),
        out_shape=jax.ShapeDtypeStruct((G_GRP, F_FEAT, HW), jnp.float32),
    )(w, g, f)


def kernel(feat_map, grp_ids, grid_ids, ins_feats):
    pad = N_PAD - N_INS
    grp_p = jnp.pad(grp_ids, (0, pad))
    gx_p = jnp.pad(grid_ids[:, 0], (0, pad))
    gy_p = jnp.pad(grid_ids[:, 1], (0, pad))
    cells = _tc_cells(grp_p, gx_p, gy_p)
    w, g = _sc_build(cells, ins_feats)
    out = _tc_select(
        w.reshape(G_GRP, 1, HW),
        g.reshape(G_GRP, HW, F_FEAT),
        feat_map.reshape(G_GRP, F_FEAT, HW),
    )
    return out.reshape(feat_map.shape)
